# Initial kernel scaffold; baseline (speedup 1.0000x reference)
import jax
import jax.numpy as jnp
from jax.experimental import pallas as pl


def kernel(emb_src, emb_tgt, W_enc, b_enc, W_dec, src_id, tgt_id):
    raise NotImplementedError("write your pallas kernel here")


# speedup vs baseline: 2.0866x; 2.0866x over previous
import functools

import jax
import jax.numpy as jnp
from jax import lax
from jax.experimental import pallas as pl
from jax.experimental.pallas import tpu as pltpu
from jax.experimental.pallas import tpu_sc as plsc

DIM = 300
VOCAB = 75000
BATCH = 16384

_INFO = plsc.get_sparse_core_info()
_NC, _NS = _INFO.num_cores, _INFO.num_subcores
_NW = _NC * _NS
_BPW = BATCH // _NW


_K1_ROWS = 1000
_K1_N = VOCAB // _K1_ROWS


def _k1_body(src_hbm, tgt_hbm, o_ref, sbuf, tbuf, ssem, tsem):
    i = pl.program_id(0)

    def _start(j, slot):
        pltpu.make_async_copy(
            src_hbm.at[pl.ds(j * _K1_ROWS, _K1_ROWS), pl.ds(256, 44)],
            sbuf.at[slot, :, pl.ds(256, 44)], ssem.at[slot]).start()
        pltpu.make_async_copy(
            tgt_hbm.at[pl.ds(j * _K1_ROWS, _K1_ROWS), pl.ds(256, 44)],
            tbuf.at[slot, :, pl.ds(256, 44)], tsem.at[slot]).start()

    @pl.when(i == 0)
    def _():
        _start(0, 0)

    @pl.when(i + 1 < _K1_N)
    def _():
        _start(i + 1, (i + 1) % 2)

    slot = i % 2
    pltpu.make_async_copy(
        src_hbm.at[pl.ds(i * _K1_ROWS, _K1_ROWS), pl.ds(256, 44)],
        sbuf.at[slot, :, pl.ds(256, 44)], ssem.at[slot]).wait()
    pltpu.make_async_copy(
        tgt_hbm.at[pl.ds(i * _K1_ROWS, _K1_ROWS), pl.ds(256, 44)],
        tbuf.at[slot, :, pl.ds(256, 44)], tsem.at[slot]).wait()
    o_ref[:, 0:44] = sbuf[slot, :, 256:300]
    o_ref[:, 64:108] = tbuf[slot, :, 256:300]


def _extract_tails(emb_src, emb_tgt):
    return pl.pallas_call(
        _k1_body,
        grid=(_K1_N,),
        in_specs=[
            pl.BlockSpec(memory_space=pl.ANY),
            pl.BlockSpec(memory_space=pl.ANY),
        ],
        out_specs=pl.BlockSpec((_K1_ROWS, 128), lambda i: (i, 0)),
        out_shape=jax.ShapeDtypeStruct((VOCAB, 128), jnp.float32),
        scratch_shapes=[
            pltpu.VMEM((2, _K1_ROWS, 300), jnp.float32),
            pltpu.VMEM((2, _K1_ROWS, 300), jnp.float32),
            pltpu.SemaphoreType.DMA((2,)),
            pltpu.SemaphoreType.DMA((2,)),
        ],
    )(emb_src, emb_tgt)


def _k2_body(emb_src, emb_tgt, tails, src_id, tgt_id, g, idx_v, rows_v, sem):
    wid = lax.axis_index("s") * _NC + lax.axis_index("c")
    start = wid * _BPW
    for t, (tab, ids) in enumerate(((emb_src, src_id), (emb_tgt, tgt_id))):
        pltpu.sync_copy(ids.at[pl.ds(start, _BPW)], idx_v)
        for j in range(2):
            pltpu.async_copy(
                tab.at[idx_v, pl.ds(j * 128, 128)], rows_v, sem).wait()
            pltpu.sync_copy(
                rows_v, g.at[t, pl.ds(start, _BPW), pl.ds(j * 128, 128)])
        pltpu.async_copy(tails.at[idx_v], rows_v, sem).wait()
        pltpu.sync_copy(rows_v, g.at[t, pl.ds(start, _BPW), pl.ds(256, 128)])


_sc_gather = functools.partial(
    pl.kernel,
    mesh=plsc.VectorSubcoreMesh(core_axis_name="c", subcore_axis_name="s"),
    out_type=jax.ShapeDtypeStruct((2, BATCH, 384), jnp.float32),
    scratch_types=[
        pltpu.VMEM((_BPW,), jnp.int32),
        pltpu.VMEM((_BPW, 128), jnp.float32),
        pltpu.SemaphoreType.DMA,
    ],
)(_k2_body)


_BM = 1024
_NTOP = BATCH // _BM


def _k3_body(g_ref, we_ref, wd_ref, b_ref, o_ref, wc_ref, b2_ref):
    i = pl.program_id(0)

    @pl.when(i == 0)
    def _():
        wc_ref[0:DIM, :] = lax.dot_general(
            we_ref[...], wd_ref[...], (((0,), (0,)), ((), ())),
            preferred_element_type=jnp.float32)
        wc_ref[DIM:384, :] = jnp.zeros((384 - DIM, DIM), jnp.float32)
        b2_ref[...] = jnp.dot(b_ref[...], wd_ref[...],
                              preferred_element_type=jnp.float32)

    @pl.when(i < _NTOP)
    def _():
        x = g_ref[0]
        col = lax.broadcasted_iota(jnp.int32, (_BM, 384), 1)
        x = jnp.where(col < DIM, x, 0.0)
        o_ref[...] = jnp.dot(x, wc_ref[...],
                             preferred_element_type=jnp.float32) + b2_ref[...]

    @pl.when(i >= _NTOP)
    def _():
        y = g_ref[0]
        o_ref[...] = jnp.concatenate([y[:, 0:256], y[:, 320:364]], axis=1)


def _tc_finish(g, W_enc, W_dec, b_enc):
    return pl.pallas_call(
        _k3_body,
        grid=(2 * _NTOP,),
        in_specs=[
            pl.BlockSpec((1, _BM, 384), lambda i: (i // _NTOP, i % _NTOP, 0)),
            pl.BlockSpec((DIM, DIM), lambda i: (0, 0)),
            pl.BlockSpec((DIM, DIM), lambda i: (0, 0)),
            pl.BlockSpec((1, DIM), lambda i: (0, 0)),
        ],
        out_specs=pl.BlockSpec((_BM, DIM), lambda i: (i, 0)),
        out_shape=jax.ShapeDtypeStruct((2 * BATCH, DIM), jnp.float32),
        scratch_shapes=[
            pltpu.VMEM((384, DIM), jnp.float32),
            pltpu.VMEM((1, DIM), jnp.float32),
        ],
    )(g, W_enc, W_dec, b_enc)


def kernel(emb_src, emb_tgt, W_enc, b_enc, W_dec, src_id, tgt_id):
    src_id = src_id.astype(jnp.int32)
    tgt_id = tgt_id.astype(jnp.int32)
    tails = _extract_tails(emb_src, emb_tgt)
    g = _sc_gather(emb_src, emb_tgt, tails, src_id, tgt_id)
    return _tc_finish(g, W_enc, W_dec, b_enc.reshape(1, DIM))
